# Initial kernel scaffold; baseline (speedup 1.0000x reference)
import jax
import jax.numpy as jnp
from jax.experimental import pallas as pl


def kernel(features, edge_index, W1, b1, W2, b2, W3, b3, W4, b4):
    raise NotImplementedError("write your pallas kernel here")


# speedup vs baseline: 1.3029x; 1.3029x over previous
import functools

import jax
import jax.numpy as jnp
from jax import lax
from jax.experimental import pallas as pl
from jax.experimental.pallas import tpu as pltpu
from jax.experimental.pallas import tpu_sc as plsc

N_NODES = 10000
NROWS = 10112
NC, NS = 2, 16
NW = NC * NS
SUB = 128
NSUB = 7
EPT = SUB * NSUB
E_PAD = EPT * NW
RPT = NROWS // NS
ZR = 64
ROW_B = 1000


def _agg_body(C, Dc, y_ref, src_ref, dst_ref, out_ref,
              src_v, dst_v, idx_v, rows_v, zeros_v, acc, gsem):
    core = lax.axis_index("c")
    sub = lax.axis_index("s")
    wid = core * NS + sub

    pltpu.sync_copy(src_ref.at[wid], src_v)
    pltpu.sync_copy(dst_ref.at[wid], dst_v)

    zero16 = jnp.zeros((16,), jnp.float32)

    def _z(r, carry):
        for i in range(Dc // 16):
            zeros_v[r, pl.ds(i * 16, 16)] = zero16
        return carry

    lax.fori_loop(0, ZR, _z, 0)

    for c in range(C):
        for j in range(RPT // ZR):
            pltpu.sync_copy(zeros_v, acc.at[pl.ds(sub * RPT + j * ZR, ZR)])
        rem = RPT % ZR
        if rem:
            pltpu.sync_copy(zeros_v.at[pl.ds(0, rem)],
                            acc.at[pl.ds(sub * RPT + (RPT // ZR) * ZR, rem)])
        if C > 1:
            for j in range(NSUB):
                for i in range(SUB // 16):
                    idx_v[j, pl.ds(i * 16, 16)] = src_v[j, pl.ds(i * 16, 16)] * C + c
        plsc.subcore_barrier()
        ivec = idx_v if C > 1 else src_v
        for j in range(NSUB):
            pltpu.async_copy(y_ref.at[ivec.at[j]], rows_v, gsem).wait()
            pltpu.sync_copy(rows_v, acc.at[dst_v.at[j]], add=True)
        plsc.subcore_barrier()
        pltpu.sync_copy(acc.at[pl.ds(sub * RPT, RPT)],
                        out_ref.at[core, c, pl.ds(sub * RPT, RPT)])


def _make_agg(C, Dc):
    mesh = plsc.VectorSubcoreMesh(core_axis_name="c", subcore_axis_name="s")
    return pl.kernel(
        functools.partial(_agg_body, C, Dc),
        out_type=jax.ShapeDtypeStruct((NC, C, NROWS, Dc), jnp.float32),
        mesh=mesh,
        scratch_types=[
            pltpu.VMEM((NSUB, SUB), jnp.int32),
            pltpu.VMEM((NSUB, SUB), jnp.int32),
            pltpu.VMEM((NSUB, SUB), jnp.int32),
            pltpu.VMEM((SUB, Dc), jnp.float32),
            pltpu.VMEM((ZR, Dc), jnp.float32),
            pltpu.VMEM_SHARED((NROWS, Dc), jnp.float32),
            pltpu.SemaphoreType.DMA,
        ],
    )


def _mm1_body(a_ref, w_ref, o_ref):
    o_ref[...] = jnp.dot(a_ref[...].astype(jnp.bfloat16), w_ref[...],
                         preferred_element_type=jnp.float32)


def _mm1(x, w_bf):
    r = x.shape[0] // ROW_B
    return pl.pallas_call(
        _mm1_body,
        grid=(r,),
        in_specs=[pl.BlockSpec((ROW_B, x.shape[1]), lambda i: (i, 0)),
                  pl.BlockSpec(w_bf.shape, lambda i: (0, 0))],
        out_specs=pl.BlockSpec((ROW_B, w_bf.shape[1]), lambda i: (i, 0)),
        out_shape=jax.ShapeDtypeStruct((x.shape[0], w_bf.shape[1]), jnp.float32),
    )(x, w_bf)


def _cmm_body(relu, p_ref, w_ref, b_ref, o_ref):
    c = pl.program_id(1)
    h = p_ref[0, 0] + p_ref[1, 0] + b_ref[0]
    if relu:
        h = jnp.maximum(h, 0.0)
    contrib = jnp.dot(h.astype(jnp.bfloat16), w_ref[0],
                      preferred_element_type=jnp.float32)

    @pl.when(c == 0)
    def _():
        o_ref[...] = contrib

    @pl.when(c != 0)
    def _():
        o_ref[...] += contrib


def _cmm(parts, w_bf, b2d, relu):
    _, C, _, Dc = parts.shape
    dout = w_bf.shape[2]
    r = N_NODES // ROW_B
    return pl.pallas_call(
        functools.partial(_cmm_body, relu),
        grid=(r, C),
        in_specs=[pl.BlockSpec((2, 1, ROW_B, Dc), lambda i, c: (0, c, i, 0)),
                  pl.BlockSpec((1, Dc, dout), lambda i, c: (c, 0, 0)),
                  pl.BlockSpec((1, 1, Dc), lambda i, c: (c, 0, 0))],
        out_specs=pl.BlockSpec((ROW_B, dout), lambda i, c: (i, 0)),
        out_shape=jax.ShapeDtypeStruct((N_NODES, dout), jnp.float32),
    )(parts, w_bf, b2d)


def _fin_body(p_ref, b_ref, o_ref):
    h = p_ref[0, 0] + p_ref[1, 0] + b_ref[...]
    o_ref[...] = jnp.tanh(h[:, :10])


def _fin(parts, b4p):
    r = N_NODES // ROW_B
    return pl.pallas_call(
        _fin_body,
        grid=(r,),
        in_specs=[pl.BlockSpec((2, 1, ROW_B, 128), lambda i: (0, 0, i, 0)),
                  pl.BlockSpec((1, 128), lambda i: (0, 0))],
        out_specs=pl.BlockSpec((ROW_B, 10), lambda i: (i, 0)),
        out_shape=jax.ShapeDtypeStruct((N_NODES, 10), jnp.float32),
    )(parts, b4p)


def kernel(features, edge_index, W1, b1, W2, b2, W3, b3, W4, b4):
    x = features.reshape(N_NODES, 32 * 32 * 3)
    src = edge_index[0].astype(jnp.int32)
    dst = edge_index[1].astype(jnp.int32)
    npad = E_PAD - src.shape[0]
    src_t = jnp.concatenate([src, jnp.zeros((npad,), jnp.int32)]).reshape(NW, NSUB, SUB)
    dst_t = jnp.concatenate([dst, jnp.full((npad,), N_NODES, jnp.int32)]).reshape(NW, NSUB, SUB)

    def prep_w(w, din_p, dout_p):
        return jnp.pad(w, ((0, din_p - w.shape[0]),
                           (0, dout_p - w.shape[1]))).astype(jnp.bfloat16)

    w1p = prep_w(W1, 3072, 1024)
    w2p = prep_w(W2, 1024, 384).reshape(8, 128, 384)
    w3p = prep_w(W3, 384, 128).reshape(3, 128, 128)
    w4p = prep_w(W4, 128, 128).reshape(1, 128, 128)
    b1p = jnp.pad(b1, (0, 1024 - 1000)).reshape(8, 1, 128)
    b2p = jnp.pad(b2, (0, 384 - 300)).reshape(3, 1, 128)
    b3p = jnp.pad(b3, (0, 128 - 20)).reshape(1, 1, 128)
    b4p = jnp.pad(b4, (0, 128 - 10)).reshape(1, 128)

    y1 = _mm1(x, w1p)
    p1 = _make_agg(8, 128)(y1.reshape(N_NODES * 8, 128), src_t, dst_t)
    y2 = _cmm(p1, w2p, b1p, relu=True)
    p2 = _make_agg(3, 128)(y2.reshape(N_NODES * 3, 128), src_t, dst_t)
    y3 = _cmm(p2, w3p, b2p, relu=True)
    p3 = _make_agg(1, 128)(y3, src_t, dst_t)
    y4 = _cmm(p3, w4p, b3p, relu=False)
    p4 = _make_agg(1, 128)(y4, src_t, dst_t)
    return _fin(p4, b4p)
